# Initial kernel scaffold; baseline (speedup 1.0000x reference)
import jax
import jax.numpy as jnp
from jax.experimental import pallas as pl


def kernel(x, y, t2v_w, t2v_b, local_table, vt_w, vt_b, space_table, given_table):
    raise NotImplementedError("write your pallas kernel here")


# speedup vs baseline: 4.6241x; 4.6241x over previous
import jax
import jax.numpy as jnp
from jax.experimental import pallas as pl

_B, _N, _MAP, _DY, _DX = 4, 512, 4, 8, 6
_D = 256
_TE = 6
_TD = _TE * _DX
_K = _N * _MAP * _DY
_KT = 2048
_NBLK = _K // _KT


def _body(x_ref, y_ref, yg_ref, t2vw_ref, t2vb_ref, local_ref, vtw_ref,
          vtb_ref, space_ref, given_ref, val_ref, space_out_ref, var_ref):
    c = pl.program_id(1)
    x = x_ref[0]
    xn = jnp.where(jnp.isnan(x), 0.0, x)
    xrep = jnp.repeat(xn, _TE, axis=1)
    xa = xrep * t2vw_ref[...] + t2vb_ref[...]
    col = jax.lax.broadcasted_iota(jnp.int32, (_N, _TD), 1)
    tv = jnp.where(col % _TE == 0, xa, jnp.sin(xa))
    t_tab = jnp.dot(tv, vtw_ref[: _TD, :],
                    preferred_element_type=jnp.float32)
    t_exp = jnp.tile(t_tab, (_KT // _N, 1))
    local_exp = jnp.repeat(local_ref[...], 32, axis=0)
    yv = y_ref[0, 0]
    ymask = jnp.isnan(yv)
    yc = jnp.where(ymask, 0.0, yv)
    yg = yg_ref[0, 0]
    gmask = jnp.isnan(yg)
    grow = jnp.where(gmask, given_ref[0:1, :], given_ref[1:2, :])
    wy = vtw_ref[_TD:_TD + 1, :]
    val_ref[0] = (t_exp + local_exp + grow + yc * wy + vtb_ref[...])
    rows = space_ref[...]
    rsel = jax.lax.broadcasted_iota(jnp.int32, (_DY, 1), 0) == c
    srow = jnp.sum(jnp.where(rsel, rows, 0.0), axis=0, keepdims=True)
    space_out_ref[0] = jnp.broadcast_to(srow, (_KT, _D))
    var_ref[0, 0] = jnp.full((1, _KT), c, jnp.int32)


def kernel(x, y, t2v_w, t2v_b, local_table, vt_w, vt_b, space_table,
           given_table):
    batch = x.shape[0]
    y_flat = y.reshape(batch, _NBLK, _KT, 1)
    yg_flat = jnp.transpose(y, (0, 1, 3, 2)).reshape(batch, _NBLK, _KT, 1)
    t2vw_f = t2v_w.reshape(1, _TD)
    t2vb_f = t2v_b.reshape(1, _TD)
    vtb_f = vt_b.reshape(1, _D)

    grid = (batch, _NBLK)
    val, space_emb, var4 = pl.pallas_call(
        _body,
        grid=grid,
        in_specs=[
            pl.BlockSpec((1, _N, _DX), lambda b, c: (b, 0, 0)),
            pl.BlockSpec((1, 1, _KT, 1), lambda b, c: (b, c, 0, 0)),
            pl.BlockSpec((1, 1, _KT, 1), lambda b, c: (b, c, 0, 0)),
            pl.BlockSpec((1, _TD), lambda b, c: (0, 0)),
            pl.BlockSpec((1, _TD), lambda b, c: (0, 0)),
            pl.BlockSpec((_KT // 32, _D), lambda b, c: (c, 0)),
            pl.BlockSpec((_TD + 1, _D), lambda b, c: (0, 0)),
            pl.BlockSpec((1, _D), lambda b, c: (0, 0)),
            pl.BlockSpec((_DY, _D), lambda b, c: (0, 0)),
            pl.BlockSpec((2, _D), lambda b, c: (0, 0)),
        ],
        out_specs=[
            pl.BlockSpec((1, _KT, _D), lambda b, c: (b, c, 0)),
            pl.BlockSpec((1, _KT, _D), lambda b, c: (b, c, 0)),
            pl.BlockSpec((1, 1, 1, _KT), lambda b, c: (b, c, 0, 0)),
        ],
        out_shape=[
            jax.ShapeDtypeStruct((batch, _K, _D), jnp.float32),
            jax.ShapeDtypeStruct((batch, _K, _D), jnp.float32),
            jax.ShapeDtypeStruct((batch, _NBLK, 1, _KT), jnp.int32),
        ],
    )(x, y_flat, yg_flat, t2vw_f, t2vb_f, local_table, vt_w, vtb_f,
      space_table, given_table)
    return (val, space_emb, var4.reshape(batch, _K))
